# speedup vs baseline: 1.0810x; 1.0810x over previous
import functools

import jax
import jax.numpy as jnp
from jax.experimental import pallas as pl

N = 10000
D = 128
EPS = 1e-5
BLK = 1000


def _dense_body(agg_ref, deg_ref, x_ref, wl_ref, wr_ref, b_ref,
                h_ref, stats_ref, *, with_stats):
    aggsum = agg_ref[0] + agg_ref[1]
    inv = 1.0 / jnp.maximum(deg_ref[:, 0:1], 1.0)
    h = (jnp.dot(aggsum * inv, wl_ref[...], preferred_element_type=jnp.float32)
         + jnp.dot(x_ref[...], wr_ref[...], preferred_element_type=jnp.float32)
         + b_ref[...])
    h_ref[...] = h
    if with_stats:
        s = jnp.sum(h, axis=0, keepdims=True)
        s2 = jnp.sum(h * h, axis=0, keepdims=True)
        st = jnp.concatenate([s, s2], axis=0)

        @pl.when(pl.program_id(0) == 0)
        def _():
            stats_ref[...] = jnp.zeros_like(stats_ref)

        stats_ref[...] += st


def _dense_stage(agg2, deg16, x, wl_t, wr_t, b, with_stats):
    grid = N // BLK
    out_shapes = [jax.ShapeDtypeStruct((N, D), jnp.float32)]
    out_specs = [pl.BlockSpec((BLK, D), lambda i: (i, 0))]
    if with_stats:
        out_shapes.append(jax.ShapeDtypeStruct((2, D), jnp.float32))
        out_specs.append(pl.BlockSpec((2, D), lambda i: (0, 0)))
    else:
        out_shapes.append(jax.ShapeDtypeStruct((1, 1), jnp.float32))
        out_specs.append(pl.BlockSpec((1, 1), lambda i: (0, 0)))
    body = functools.partial(_dense_body, with_stats=with_stats)
    res = pl.pallas_call(
        body,
        grid=(grid,),
        in_specs=[
            pl.BlockSpec((2, BLK, D), lambda i: (0, i, 0)),
            pl.BlockSpec((BLK, 16), lambda i: (i, 0)),
            pl.BlockSpec((BLK, D), lambda i: (i, 0)),
            pl.BlockSpec((D, D), lambda i: (0, 0)),
            pl.BlockSpec((D, D), lambda i: (0, 0)),
            pl.BlockSpec((1, D), lambda i: (0, 0)),
        ],
        out_specs=out_specs,
        out_shape=out_shapes,
    )(agg2, deg16, x, wl_t, wr_t, b)
    return res


def _bn_relu_body(h_ref, scale_ref, shift_ref, o_ref):
    o_ref[...] = jnp.maximum(h_ref[...] * scale_ref[...] + shift_ref[...], 0.0)


def _bn_relu(h, scale, shift):
    grid = N // BLK
    return pl.pallas_call(
        _bn_relu_body,
        grid=(grid,),
        in_specs=[
            pl.BlockSpec((BLK, D), lambda i: (i, 0)),
            pl.BlockSpec((1, D), lambda i: (0, 0)),
            pl.BlockSpec((1, D), lambda i: (0, 0)),
        ],
        out_specs=pl.BlockSpec((BLK, D), lambda i: (i, 0)),
        out_shape=jax.ShapeDtypeStruct((N, D), jnp.float32),
    )(h, scale, shift)


def _aggregate(x, src, dst):
    msg = jnp.take(x, src, axis=0)
    agg = jax.ops.segment_sum(msg, dst, num_segments=N)
    return jnp.stack([agg, jnp.zeros_like(agg)], axis=0)


def kernel(x, edge_index, W_l0, b_l0, W_r0, g0, be0,
           W_l1, b_l1, W_r1, g1, be1, W_l2, b_l2, W_r2):
    src = edge_index[0]
    dst = edge_index[1]
    deg = jax.ops.segment_sum(jnp.ones((src.shape[0],), jnp.float32), dst,
                              num_segments=N)
    deg16 = jnp.broadcast_to(deg[:, None], (N, 16))

    h = x
    params = [(W_l0, b_l0, W_r0, g0, be0), (W_l1, b_l1, W_r1, g1, be1),
              (W_l2, b_l2, W_r2, None, None)]
    for li, (W_l, b_l, W_r, g, be) in enumerate(params):
        agg2 = _aggregate(h, src, dst)
        with_stats = li < 2
        h, stats = _dense_stage(agg2, deg16, h, W_l.T, W_r.T,
                                b_l.reshape(1, D), with_stats)
        if with_stats:
            mean = stats[0] / N
            var = stats[1] / N - mean * mean
            scale = g * jax.lax.rsqrt(var + EPS)
            shift = be - mean * scale
            h = _bn_relu(h, scale.reshape(1, D), shift.reshape(1, D))
    return h


# speedup vs baseline: 11.1879x; 10.3498x over previous
import functools

import jax
import jax.numpy as jnp
from jax import lax
from jax.experimental import pallas as pl
from jax.experimental.pallas import tpu as pltpu
from jax.experimental.pallas import tpu_sc as plsc

N = 10000
D = 128
E = 320000
EPS = 1e-5
BLK = 1000

NC = 2
NS = 16
NW = NC * NS
EPW = E // NW
C = 128
NCH = 80
EPWP = NCH * C
NP = 10240
RPS = NP // NS

_SC_MESH = plsc.VectorSubcoreMesh(core_axis_name="c", subcore_axis_name="s")


def _fill(ref, nrows, value):
    def row(i, _):
        for k in range(8):
            ref[i, pl.ds(k * 16, 16)] = jnp.full((16,), value, jnp.float32)
        return 0

    lax.fori_loop(0, nrows, row, 0)


def _make_agg_body(with_gather):
    def body(*args):
        if with_gather:
            (x_hbm, src_hbm, dst_hbm, agg_hbm,
             isrc0, isrc1, idst0, idst1, rows0, rows1, agg_sh,
             ss0, ss1, sd0, sd1, sr0, sr1) = args
            isrc = (isrc0, isrc1)
            ssem = (ss0, ss1)
        else:
            (dst_hbm, agg_hbm,
             idst0, idst1, rows0, rows1, agg_sh, sd0, sd1) = args
        core = lax.axis_index("c")
        sub = lax.axis_index("s")
        w = core * NS + sub

        idst = (idst0, idst1)
        rows = (rows0, rows1)
        dsem = (sd0, sd1)
        if with_gather:
            rsem = (sr0, sr1)

        _fill(rows0, C, 0.0)
        for j in range(RPS // C):
            pltpu.sync_copy(rows0, agg_sh.at[pl.ds(sub * RPS + j * C, C)])
        plsc.subcore_barrier()

        if not with_gather:
            _fill(rows0, C, 1.0)
            _fill(rows1, C, 1.0)

        for b in range(2):
            pltpu.make_async_copy(dst_hbm.at[w, b], idst[b], dsem[b]).start()
            if with_gather:
                cp = pltpu.make_async_copy(src_hbm.at[w, b], isrc[b], ssem[b])
                cp.start()
                cp.wait()
                pltpu.make_async_copy(x_hbm.at[isrc[b]], rows[b], rsem[b]).start()

        def step(g, _):
            for b in range(2):
                j = 2 * g + b
                pltpu.make_async_copy(dst_hbm.at[w, j], idst[b], dsem[b]).wait()
                nxt = j + 2
                if with_gather:
                    pltpu.make_async_copy(
                        x_hbm.at[isrc[b]], rows[b], rsem[b]).wait()

                    @pl.when(nxt < NCH)
                    def _():
                        pltpu.make_async_copy(
                            src_hbm.at[w, nxt], isrc[b], ssem[b]).start()

                pltpu.sync_copy(rows[b], agg_sh.at[idst[b]], add=True)

                @pl.when(nxt < NCH)
                def _():
                    if with_gather:
                        pltpu.make_async_copy(
                            src_hbm.at[w, nxt], isrc[b], ssem[b]).wait()
                        pltpu.make_async_copy(
                            x_hbm.at[isrc[b]], rows[b], rsem[b]).start()
                    pltpu.make_async_copy(
                        dst_hbm.at[w, nxt], idst[b], dsem[b]).start()
            return 0

        lax.fori_loop(0, NCH // 2, step, 0)
        plsc.subcore_barrier()

        for j in range(RPS // C):
            buf = rows[j % 2]
            pltpu.sync_copy(agg_sh.at[pl.ds(sub * RPS + j * C, C)], buf)
            pltpu.sync_copy(buf, agg_hbm.at[core, pl.ds(sub * RPS + j * C, C)])

    return body


_GATHER_SCRATCH = [
    pltpu.VMEM((C,), jnp.int32),
    pltpu.VMEM((C,), jnp.int32),
    pltpu.VMEM((C,), jnp.int32),
    pltpu.VMEM((C,), jnp.int32),
    pltpu.VMEM((C, D), jnp.float32),
    pltpu.VMEM((C, D), jnp.float32),
    pltpu.VMEM_SHARED((NP, D), jnp.float32),
    pltpu.SemaphoreType.DMA,
    pltpu.SemaphoreType.DMA,
    pltpu.SemaphoreType.DMA,
    pltpu.SemaphoreType.DMA,
    pltpu.SemaphoreType.DMA,
    pltpu.SemaphoreType.DMA,
]


def _sc_aggregate(x, src3d, dst3d):
    return pl.kernel(
        _make_agg_body(True),
        out_type=jax.ShapeDtypeStruct((NC, NP, D), jnp.float32),
        mesh=_SC_MESH,
        scratch_types=_GATHER_SCRATCH,
    )(x, src3d, dst3d)


def _sc_degrees(dst3d):
    return pl.kernel(
        _make_agg_body(False),
        out_type=jax.ShapeDtypeStruct((NC, NP, D), jnp.float32),
        mesh=_SC_MESH,
        scratch_types=[
            pltpu.VMEM((C,), jnp.int32),
            pltpu.VMEM((C,), jnp.int32),
            pltpu.VMEM((C, D), jnp.float32),
            pltpu.VMEM((C, D), jnp.float32),
            pltpu.VMEM_SHARED((NP, D), jnp.float32),
            pltpu.SemaphoreType.DMA,
            pltpu.SemaphoreType.DMA,
        ],
    )(dst3d)


def _dense_body(agg_ref, deg_ref, x_ref, wl_ref, wr_ref, b_ref,
                h_ref, stats_ref, *, with_stats):
    aggsum = agg_ref[0] + agg_ref[1]
    deg = deg_ref[0, :, 0:1] + deg_ref[1, :, 0:1]
    inv = 1.0 / jnp.maximum(deg, 1.0)
    h = (jnp.dot(aggsum * inv, wl_ref[...], preferred_element_type=jnp.float32)
         + jnp.dot(x_ref[...], wr_ref[...], preferred_element_type=jnp.float32)
         + b_ref[...])
    h_ref[...] = h
    if with_stats:
        s = jnp.sum(h, axis=0, keepdims=True)
        s2 = jnp.sum(h * h, axis=0, keepdims=True)
        st = jnp.concatenate([s, s2], axis=0)

        @pl.when(pl.program_id(0) == 0)
        def _():
            stats_ref[...] = jnp.zeros_like(stats_ref)

        stats_ref[...] += st


def _dense_stage(agg2, deg2, x, wl_t, wr_t, b, with_stats):
    grid = N // BLK
    out_shapes = [jax.ShapeDtypeStruct((N, D), jnp.float32)]
    out_specs = [pl.BlockSpec((BLK, D), lambda i: (i, 0))]
    if with_stats:
        out_shapes.append(jax.ShapeDtypeStruct((2, D), jnp.float32))
        out_specs.append(pl.BlockSpec((2, D), lambda i: (0, 0)))
    else:
        out_shapes.append(jax.ShapeDtypeStruct((1, 1), jnp.float32))
        out_specs.append(pl.BlockSpec((1, 1), lambda i: (0, 0)))
    body = functools.partial(_dense_body, with_stats=with_stats)
    res = pl.pallas_call(
        body,
        grid=(grid,),
        in_specs=[
            pl.BlockSpec((2, BLK, D), lambda i: (0, i, 0)),
            pl.BlockSpec((2, BLK, D), lambda i: (0, i, 0)),
            pl.BlockSpec((BLK, D), lambda i: (i, 0)),
            pl.BlockSpec((D, D), lambda i: (0, 0)),
            pl.BlockSpec((D, D), lambda i: (0, 0)),
            pl.BlockSpec((1, D), lambda i: (0, 0)),
        ],
        out_specs=out_specs,
        out_shape=out_shapes,
    )(agg2, deg2, x, wl_t, wr_t, b)
    return res


def _bn_relu_body(h_ref, scale_ref, shift_ref, o_ref):
    o_ref[...] = jnp.maximum(h_ref[...] * scale_ref[...] + shift_ref[...], 0.0)


def _bn_relu(h, scale, shift):
    grid = N // BLK
    return pl.pallas_call(
        _bn_relu_body,
        grid=(grid,),
        in_specs=[
            pl.BlockSpec((BLK, D), lambda i: (i, 0)),
            pl.BlockSpec((1, D), lambda i: (0, 0)),
            pl.BlockSpec((1, D), lambda i: (0, 0)),
        ],
        out_specs=pl.BlockSpec((BLK, D), lambda i: (i, 0)),
        out_shape=jax.ShapeDtypeStruct((N, D), jnp.float32),
    )(h, scale, shift)


def kernel(x, edge_index, W_l0, b_l0, W_r0, g0, be0,
           W_l1, b_l1, W_r1, g1, be1, W_l2, b_l2, W_r2):
    npad = EPWP - EPW
    src_w = edge_index[0].reshape(NW, EPW)
    dst_w = edge_index[1].reshape(NW, EPW)
    pad_src = jnp.broadcast_to(jnp.arange(npad, dtype=jnp.int32) % N,
                               (NW, npad))
    pad_dst = jnp.broadcast_to(N + (jnp.arange(npad, dtype=jnp.int32)
                                    % (NP - N)), (NW, npad))
    src3d = jnp.concatenate([src_w, pad_src], 1).reshape(NW, NCH, C)
    dst3d = jnp.concatenate([dst_w, pad_dst], 1).reshape(NW, NCH, C)
    deg2 = _sc_degrees(dst3d)

    h = x
    params = [(W_l0, b_l0, W_r0, g0, be0), (W_l1, b_l1, W_r1, g1, be1),
              (W_l2, b_l2, W_r2, None, None)]
    for li, (W_l, b_l, W_r, g, be) in enumerate(params):
        agg2 = _sc_aggregate(h, src3d, dst3d)
        with_stats = li < 2
        h, stats = _dense_stage(agg2, deg2, h, W_l.T, W_r.T,
                                b_l.reshape(1, D), with_stats)
        if with_stats:
            mean = stats[0] / N
            var = stats[1] / N - mean * mean
            scale = g * jax.lax.rsqrt(var + EPS)
            shift = be - mean * scale
            h = _bn_relu(h, scale.reshape(1, D), shift.reshape(1, D))
    return h
